# Initial kernel scaffold; baseline (speedup 1.0000x reference)
import jax
import jax.numpy as jnp
from jax.experimental import pallas as pl


def kernel(x, random_projection, codebook):
    raise NotImplementedError("write your pallas kernel here")


# speedup vs baseline: 1.0099x; 1.0099x over previous
import functools

import jax
import jax.numpy as jnp
from jax.experimental import pallas as pl


def _rpq_kernel(x_ref, rp_ref, cbt_ref, out_ref, *, k_chunk):
    t = x_ref.shape[0]
    k = cbt_ref.shape[1]
    proj = jnp.dot(x_ref[...], rp_ref[...],
                   preferred_element_type=jnp.float32)
    nsq = jnp.sum(proj * proj, axis=1, keepdims=True)
    norm = nsq * jax.lax.rsqrt(nsq)
    norm = jnp.where(nsq == 0.0, 0.0, norm)
    inv = pl.reciprocal(jnp.maximum(norm, 1e-12), approx=True)
    nx = proj * inv
    b2 = jnp.sum(nx * nx, axis=1, keepdims=True)

    def body(j, carry):
        best, bidx = carry
        cbc = cbt_ref[:, pl.ds(j * k_chunk, k_chunk)]
        a2c = jnp.sum(cbc * cbc, axis=0, keepdims=True)
        scores = jnp.dot(nx, cbc, preferred_element_type=jnp.float32)
        d2 = (a2c + b2) - 2.0 * scores
        d2 = jnp.maximum(d2, 1e-12)
        dist = d2 * jax.lax.rsqrt(d2)
        cmin = jnp.min(dist, axis=1, keepdims=True)
        iota = jax.lax.broadcasted_iota(jnp.int32, dist.shape, 1)
        carg = jnp.min(jnp.where(dist == cmin, iota, k), axis=1,
                       keepdims=True) + j * k_chunk
        take = cmin < best
        return jnp.where(take, cmin, best), jnp.where(take, carg, bidx)

    best0 = jnp.full((t, 1), jnp.inf, jnp.float32)
    bidx0 = jnp.zeros((t, 1), jnp.int32)
    _, bidx = jax.lax.fori_loop(0, k // k_chunk, body, (best0, bidx0))
    out_ref[...] = bidx.reshape(t)


@jax.jit
def kernel(x, random_projection, codebook):
    b, n, d = x.shape
    bn = b * n
    k, e = codebook.shape
    T = 1024
    KC = 512
    flat = x.reshape(bn, d)
    out = pl.pallas_call(
        functools.partial(_rpq_kernel, k_chunk=KC),
        grid=(bn // T,),
        in_specs=[
            pl.BlockSpec((T, d), lambda i: (i, 0)),
            pl.BlockSpec((d, e), lambda i: (0, 0)),
            pl.BlockSpec((e, k), lambda i: (0, 0)),
        ],
        out_specs=pl.BlockSpec((T,), lambda i: (i,)),
        out_shape=jax.ShapeDtypeStruct((bn,), jnp.int32),
    )(flat, random_projection, codebook.T)
    return out.reshape(b, n)


# speedup vs baseline: 1.3063x; 1.2935x over previous
import functools

import jax
import jax.numpy as jnp
from jax.experimental import pallas as pl


def _rpq_kernel(x_ref, rp_ref, cbt_ref, out_ref, *, k_chunk):
    t = x_ref.shape[0]
    e, k = cbt_ref.shape
    proj = jnp.dot(x_ref[...], rp_ref[...],
                   preferred_element_type=jnp.float32)
    nsq = jnp.sum(proj * proj, axis=1, keepdims=True)
    norm = jnp.sqrt(nsq)
    inv = 1.0 / jnp.maximum(norm, 1e-12)
    nx = proj * inv
    b2 = jnp.sum(nx * nx, axis=1, keepdims=True)

    nx_aug = jnp.concatenate([nx, jnp.ones((t, 1), jnp.float32), b2],
                             axis=1)

    def body(j, carry):
        best, bidx = carry
        cbc = cbt_ref[:, pl.ds(j * k_chunk, k_chunk)]
        a2c = jnp.sum(cbc * cbc, axis=0, keepdims=True)
        wc = jnp.concatenate(
            [cbc * -2.0, a2c, jnp.ones((1, k_chunk), jnp.float32)],
            axis=0)
        d2 = jnp.dot(nx_aug, wc, preferred_element_type=jnp.float32)
        cmin = jnp.min(d2, axis=1, keepdims=True)
        iota = jax.lax.broadcasted_iota(jnp.int32, d2.shape, 1)
        carg = jnp.min(jnp.where(d2 == cmin, iota, k), axis=1,
                       keepdims=True) + j * k_chunk
        take = cmin < best
        return jnp.where(take, cmin, best), jnp.where(take, carg, bidx)

    best0 = jnp.full((t, 1), jnp.inf, jnp.float32)
    bidx0 = jnp.zeros((t, 1), jnp.int32)
    _, bidx = jax.lax.fori_loop(0, k // k_chunk, body, (best0, bidx0))
    out_ref[...] = bidx.reshape(t)


@jax.jit
def kernel(x, random_projection, codebook):
    b, n, d = x.shape
    bn = b * n
    k, e = codebook.shape
    T = 1024
    KC = 512
    flat = x.reshape(bn, d)
    out = pl.pallas_call(
        functools.partial(_rpq_kernel, k_chunk=KC),
        grid=(bn // T,),
        in_specs=[
            pl.BlockSpec((T, d), lambda i: (i, 0)),
            pl.BlockSpec((d, e), lambda i: (0, 0)),
            pl.BlockSpec((e, k), lambda i: (0, 0)),
        ],
        out_specs=pl.BlockSpec((T,), lambda i: (i,)),
        out_shape=jax.ShapeDtypeStruct((bn,), jnp.int32),
    )(flat, random_projection, codebook.T)
    return out.reshape(b, n)
